# baseline (speedup 1.0000x reference)
import jax
import jax.numpy as jnp
from jax.experimental import pallas as pl
from jax.experimental.pallas import tpu as pltpu

_G = 2
_NSL = 2


def _stream_kernel(mask_ref, w_ref, b_ref, emb_ref, emb_out_ref, logits_ref,
                   buf, lgbuf, sem_in, sem_out, sem_lg):
    n_tiles, S, D = emb_ref.shape
    L = w_ref.shape[0]
    C = w_ref.shape[2]
    n_chunks = n_tiles // _G

    def in_copy(i, slot):
        return pltpu.make_async_copy(
            emb_ref.at[pl.ds(i * _G, _G)], buf.at[slot], sem_in.at[slot])

    def out_copy(i, slot):
        return pltpu.make_async_copy(
            buf.at[slot], emb_out_ref.at[pl.ds(i * _G, _G)], sem_out.at[slot])

    def lg_copy(i, slot):
        return pltpu.make_async_copy(
            lgbuf.at[slot], logits_ref.at[pl.ds(i * _G, _G)], sem_lg.at[slot])

    in_copy(0, 0).start()

    def body(i, _):
        slot = jax.lax.rem(i, _NSL)
        in_copy(i, slot).wait()
        out_copy(i, slot).start()

        @pl.when(i >= _NSL)
        def _():
            lg_copy(i - _NSL, slot).wait()

        for g in range(_G):
            tile = i * _G + g
            lyr = jax.lax.rem(tile, L)
            bidx = jax.lax.div(tile, L)
            y = jnp.dot(buf[slot, g], w_ref[lyr],
                        preferred_element_type=jnp.float32)
            lgbuf[slot, g] = y + b_ref[lyr] + mask_ref[bidx]
        lg_copy(i, slot).start()

        out_copy(i, slot).wait()

        @pl.when(i + 1 < n_chunks)
        def _():
            in_copy(i + 1, jax.lax.rem(i + 1, _NSL)).start()
        return 0

    jax.lax.fori_loop(0, n_chunks, body, 0)

    for c in range(max(0, n_chunks - _NSL), n_chunks):
        lg_copy(c, c % _NSL).wait()


@jax.jit
def _run(emb_flat, mask, W, b3):
    T, S, D = emb_flat.shape
    L, _, C = W.shape

    emb_out, logits = pl.pallas_call(
        _stream_kernel,
        in_specs=[
            pl.BlockSpec(memory_space=pltpu.MemorySpace.VMEM),
            pl.BlockSpec(memory_space=pltpu.MemorySpace.VMEM),
            pl.BlockSpec(memory_space=pltpu.MemorySpace.VMEM),
            pl.BlockSpec(memory_space=pltpu.MemorySpace.HBM),
        ],
        out_specs=[
            pl.BlockSpec(memory_space=pltpu.MemorySpace.HBM),
            pl.BlockSpec(memory_space=pltpu.MemorySpace.HBM),
        ],
        out_shape=[
            jax.ShapeDtypeStruct((T, S, D), jnp.float32),
            jax.ShapeDtypeStruct((T, S, C), jnp.float32),
        ],
        scratch_shapes=[
            pltpu.VMEM((_NSL, _G, S, D), jnp.float32),
            pltpu.VMEM((_NSL, _G, S, C), jnp.float32),
            pltpu.SemaphoreType.DMA((_NSL,)),
            pltpu.SemaphoreType.DMA((_NSL,)),
            pltpu.SemaphoreType.DMA((_NSL,)),
        ],
    )(mask, W, b3, emb_flat)
    return emb_out, logits


def kernel(emb_sentences, att_sentences, W, b):
    B, L, S, D = emb_sentences.shape
    C = W.shape[-1]
    mask = jnp.where(att_sentences, 0.0, -jnp.inf).astype(jnp.float32)
    mask = mask.reshape(B, S, 1)
    b3 = b.reshape(L, 1, C)
    emb_flat = emb_sentences.reshape(B * L, S, D)
    emb_out, logits = _run(emb_flat, mask, W, b3)
    return (emb_out.reshape(B, L, S, D), att_sentences,
            logits.reshape(B, L, S, C))


# speedup vs baseline: 1.1064x; 1.1064x over previous
import jax
import jax.numpy as jnp
from jax.experimental import pallas as pl
from jax.experimental.pallas import tpu as pltpu


def _fused_kernel(mask_ref, w_ref, b_ref, emb_ref, emb_out_ref, logits_ref):
    x = emb_ref[0]
    emb_out_ref[0] = x
    y = jnp.dot(x.astype(jnp.bfloat16), w_ref[0],
                preferred_element_type=jnp.float32)
    logits_ref[0] = y + b_ref[0] + mask_ref[0]


@jax.jit
def _run(emb_flat, mask3, W16, b3):
    T, S, D = emb_flat.shape
    L, _, C = W16.shape
    BL = L

    emb_out, logits = pl.pallas_call(
        _fused_kernel,
        grid=(T,),
        in_specs=[
            pl.BlockSpec((1, S, 1), lambda i: (i // BL, 0, 0)),
            pl.BlockSpec((1, D, C), lambda i: (i % BL, 0, 0)),
            pl.BlockSpec((1, 1, C), lambda i: (i % BL, 0, 0)),
            pl.BlockSpec((1, S, D), lambda i: (i, 0, 0)),
        ],
        out_specs=[
            pl.BlockSpec((1, S, D), lambda i: (i, 0, 0)),
            pl.BlockSpec((1, S, C), lambda i: (i, 0, 0)),
        ],
        out_shape=[
            jax.ShapeDtypeStruct((T, S, D), jnp.float32),
            jax.ShapeDtypeStruct((T, S, C), jnp.float32),
        ],
    )(mask3, W16, b3, emb_flat)
    return emb_out, logits


def kernel(emb_sentences, att_sentences, W, b):
    B, L, S, D = emb_sentences.shape
    C = W.shape[-1]
    mask = jnp.where(att_sentences, 0.0, -jnp.inf).astype(jnp.float32)
    mask = mask.reshape(B, S, 1)
    b3 = b.reshape(L, 1, C)
    W16 = W.astype(jnp.bfloat16)
    emb_flat = emb_sentences.reshape(B * L, S, D)
    emb_out, logits = _run(emb_flat, mask, W16, b3)
    return (emb_out.reshape(B, L, S, D), att_sentences,
            logits.reshape(B, L, S, C))
